# baseline (speedup 1.0000x reference)
import functools

import jax
import jax.numpy as jnp
from jax.experimental import pallas as pl

_F32 = jnp.float32
_BF16 = jnp.bfloat16
_FP8 = jnp.float8_e4m3fn
_ADJ_SCALE = 4096.0
_INV_ADJ_SCALE = 1.0 / 4096.0


def _dot(a, b, tb=False):
    dn = (((1,), (1,) if tb else (0,)), ((), ()))
    return jax.lax.dot_general(a, b, dn, preferred_element_type=_F32)


def _mm_kernel(x_ref, w_ref, o_ref):
    o_ref[...] = _dot(x_ref[...], w_ref[...]).astype(o_ref.dtype)


def _mm(x, w, bm, out_dtype=_BF16):
    m, kdim = x.shape
    bm = min(bm, m)
    n = w.shape[1]
    return pl.pallas_call(
        _mm_kernel,
        grid=(m // bm,),
        in_specs=[
            pl.BlockSpec((bm, kdim), lambda i: (i, 0)),
            pl.BlockSpec((kdim, n), lambda i: (0, 0)),
        ],
        out_specs=pl.BlockSpec((bm, n), lambda i: (i, 0)),
        out_shape=jax.ShapeDtypeStruct((m, n), out_dtype),
    )(x, w)


def _gcn_cast_kernel(adj_ref, x_ref, w0_ref, b_ref, wn_ref, o_ref, a8_ref,
                     sup_ref):
    @pl.when(pl.program_id(0) == 0)
    def _():
        sup_ref[...] = _dot(x_ref[...].astype(_BF16),
                            w0_ref[...]).astype(_FP8)

    a = (adj_ref[...] * _ADJ_SCALE).astype(_FP8)
    a8_ref[...] = a
    acc = _dot(a, sup_ref[...]) * _INV_ADJ_SCALE
    z = jax.nn.sigmoid(acc + b_ref[...])
    o_ref[...] = _dot(z.astype(_BF16), wn_ref[...]).astype(o_ref.dtype)


def _gcn_cast(adj, x, w0, b, w_next, bm=1024, out_dtype=_FP8):
    from jax.experimental.pallas import tpu as pltpu
    m, kdim = adj.shape
    bm = min(bm, m)
    nfeat = x.shape[1]
    n = w0.shape[1]
    n_out = w_next.shape[1]
    return pl.pallas_call(
        _gcn_cast_kernel,
        grid=(m // bm,),
        in_specs=[
            pl.BlockSpec((bm, kdim), lambda i: (i, 0)),
            pl.BlockSpec((m, nfeat), lambda i: (0, 0)),
            pl.BlockSpec((nfeat, n), lambda i: (0, 0)),
            pl.BlockSpec((1, n), lambda i: (0, 0)),
            pl.BlockSpec((n, n_out), lambda i: (0, 0)),
        ],
        out_specs=[
            pl.BlockSpec((bm, n_out), lambda i: (i, 0)),
            pl.BlockSpec((bm, kdim), lambda i: (i, 0)),
        ],
        out_shape=[
            jax.ShapeDtypeStruct((m, n_out), out_dtype),
            jax.ShapeDtypeStruct((m, kdim), _FP8),
        ],
        scratch_shapes=[pltpu.VMEM((m, n), _FP8)],
    )(adj, x, w0, b.reshape(1, n), w_next)


def _gcn_kernel(adj_ref, sup_ref, b_ref, o_ref):
    z = _dot(adj_ref[...], sup_ref[...]) * _INV_ADJ_SCALE + b_ref[...]
    o_ref[...] = jax.nn.sigmoid(z).astype(o_ref.dtype)


def _gcn_proj_kernel(adj_ref, sup_ref, b_ref, wn_ref, o_ref):
    acc = _dot(adj_ref[...], sup_ref[...]) * _INV_ADJ_SCALE
    z = jax.nn.sigmoid(acc + b_ref[...])
    o_ref[...] = _dot(z.astype(_BF16), wn_ref[...]).astype(o_ref.dtype)


def _gcn(adj, sup, b, w_next=None, bm=1024, out_dtype=_BF16):
    m, kdim = adj.shape
    bm = min(bm, m)
    n = sup.shape[1]
    b2 = b.reshape(1, n)
    if w_next is None:
        kern, n_out, ops = _gcn_kernel, n, (adj, sup, b2)
        extra = []
    else:
        n_out = w_next.shape[1]
        kern, ops = _gcn_proj_kernel, (adj, sup, b2, w_next)
        extra = [pl.BlockSpec((n, n_out), lambda i: (0, 0))]
    return pl.pallas_call(
        kern,
        grid=(m // bm,),
        in_specs=[
            pl.BlockSpec((bm, kdim), lambda i: (i, 0)),
            pl.BlockSpec((kdim, n), lambda i: (0, 0)),
            pl.BlockSpec((1, n), lambda i: (0, 0)),
        ] + extra,
        out_specs=pl.BlockSpec((bm, n_out), lambda i: (i, 0)),
        out_shape=jax.ShapeDtypeStruct((m, n_out), out_dtype),
    )(*ops)


def _gcn_add_kernel(adj_ref, sa_ref, sb_ref, b_ref, o_ref):
    sup = (sa_ref[...].astype(_F32) + sb_ref[...]).astype(_FP8)
    z = _dot(adj_ref[...], sup) * _INV_ADJ_SCALE + b_ref[...]
    o_ref[...] = jax.nn.sigmoid(z).astype(o_ref.dtype)


def _gcn_add(adj, sup_a, sup_b, b, bm=1024):
    m, kdim = adj.shape
    bm = min(bm, m)
    n = sup_a.shape[1]
    return pl.pallas_call(
        _gcn_add_kernel,
        grid=(m // bm,),
        in_specs=[
            pl.BlockSpec((bm, kdim), lambda i: (i, 0)),
            pl.BlockSpec((kdim, n), lambda i: (0, 0)),
            pl.BlockSpec((kdim, n), lambda i: (0, 0)),
            pl.BlockSpec((1, n), lambda i: (0, 0)),
        ],
        out_specs=pl.BlockSpec((bm, n), lambda i: (i, 0)),
        out_shape=jax.ShapeDtypeStruct((m, n), _F32),
    )(adj, sup_a, sup_b, b.reshape(1, n))


def _gcn_proj2_kernel(adj_ref, sup_ref, b_ref, w1_ref, w2_ref, o1_ref,
                      o2_ref, *, n):
    acc = _dot(adj_ref[...], sup_ref[...]) * _INV_ADJ_SCALE
    z = jax.nn.sigmoid(acc + b_ref[...])
    zb = z.astype(_BF16)
    o1_ref[...] = _dot(zb[:, :n], w1_ref[...]).astype(o1_ref.dtype)
    o2_ref[...] = _dot(zb[:, n:], w2_ref[...]).astype(o2_ref.dtype)


def _gcn_proj2(adj, sup, b, w1, w2, bm=1024):
    m, kdim = adj.shape
    bm = min(bm, m)
    n2 = sup.shape[1]
    n = n2 // 2
    n_out = w1.shape[1]
    kern = functools.partial(_gcn_proj2_kernel, n=n)
    return pl.pallas_call(
        kern,
        grid=(m // bm,),
        in_specs=[
            pl.BlockSpec((bm, kdim), lambda i: (i, 0)),
            pl.BlockSpec((kdim, n2), lambda i: (0, 0)),
            pl.BlockSpec((1, n2), lambda i: (0, 0)),
            pl.BlockSpec((n, n_out), lambda i: (0, 0)),
            pl.BlockSpec((n, n_out), lambda i: (0, 0)),
        ],
        out_specs=[
            pl.BlockSpec((bm, n_out), lambda i: (i, 0)),
            pl.BlockSpec((bm, n_out), lambda i: (i, 0)),
        ],
        out_shape=[
            jax.ShapeDtypeStruct((m, n_out), _BF16),
            jax.ShapeDtypeStruct((m, n_out), _BF16),
        ],
    )(adj, sup, b.reshape(1, n2), w1, w2)


def _gcn2_kernel(a1_ref, a2_ref, s1_ref, s2_ref, b_ref, wn_ref,
                 o1_ref, o2_ref):
    g = pl.program_id(0)

    @pl.when(g == 0)
    def _():
        acc = _dot(a1_ref[...], s1_ref[...]) * _INV_ADJ_SCALE
        z = jax.nn.sigmoid(acc + b_ref[...])
        o1_ref[...] = _dot(z.astype(_BF16), wn_ref[...]).astype(o1_ref.dtype)

    @pl.when(g == 1)
    def _():
        acc = _dot(a2_ref[...], s2_ref[...]) * _INV_ADJ_SCALE
        z = jax.nn.sigmoid(acc + b_ref[...])
        o2_ref[...] = _dot(z.astype(_BF16), wn_ref[...]).astype(o2_ref.dtype)


def _gcn2(a1, a2, s1, s2, b, w_next, bm=1024, out_dtype=_FP8):
    m, kdim = a1.shape
    bm = min(bm, m)
    nblk = m // bm
    last = nblk - 1
    n = s1.shape[1]
    n_out = w_next.shape[1]
    return pl.pallas_call(
        _gcn2_kernel,
        grid=(2, nblk),
        in_specs=[
            pl.BlockSpec((bm, kdim), lambda g, i: (jnp.where(g == 0, i, last), 0)),
            pl.BlockSpec((bm, kdim), lambda g, i: (jnp.where(g == 0, 0, i), 0)),
            pl.BlockSpec((kdim, n), lambda g, i: (0, 0)),
            pl.BlockSpec((kdim, n), lambda g, i: (0, 0)),
            pl.BlockSpec((1, n), lambda g, i: (0, 0)),
            pl.BlockSpec((n, n_out), lambda g, i: (0, 0)),
        ],
        out_specs=[
            pl.BlockSpec((bm, n_out), lambda g, i: (jnp.where(g == 0, i, last), 0)),
            pl.BlockSpec((bm, n_out), lambda g, i: (jnp.where(g == 0, 0, i), 0)),
        ],
        out_shape=[
            jax.ShapeDtypeStruct((m, n_out), out_dtype),
            jax.ShapeDtypeStruct((m, n_out), out_dtype),
        ],
    )(a1, a2, s1, s2, b.reshape(1, n), w_next)


def _final2_kernel(a1_ref, a2_ref, z1_ref, v2_ref, o2t_ref, b_ref,
                   r1_ref, r2_ref):
    g = pl.program_id(0)

    @pl.when(g == 0)
    def _():
        acc = _dot(a1_ref[...], z1_ref[...]) * _INV_ADJ_SCALE
        r1_ref[...] = jax.nn.sigmoid(acc + b_ref[...])

    @pl.when(g == 1)
    def _():
        a2b = (a2_ref[...].astype(_F32) * _INV_ADJ_SCALE).astype(_BF16)
        acc = _dot(a2b, v2_ref[...]) + _dot(a2b, o2t_ref[...].astype(_BF16),
                                            tb=True)
        r2_ref[...] = jax.nn.sigmoid(acc + b_ref[...])


def _final2(a1, a2, z1, p2, o2t, b, d, nc, bm=1024):
    m, kdim = a1.shape
    bm = min(bm, m)
    nblk = m // bm
    last = nblk - 1
    voff = (d + nc) // nc - 1
    return pl.pallas_call(
        _final2_kernel,
        grid=(2, nblk),
        in_specs=[
            pl.BlockSpec((bm, kdim), lambda g, i: (jnp.where(g == 0, i, last), 0)),
            pl.BlockSpec((bm, kdim), lambda g, i: (jnp.where(g == 0, 0, i), 0)),
            pl.BlockSpec((kdim, nc), lambda g, i: (0, 0)),
            pl.BlockSpec((kdim, nc), lambda g, i: (0, voff)),
            pl.BlockSpec((nc, kdim), lambda g, i: (0, 0)),
            pl.BlockSpec((1, nc), lambda g, i: (0, 0)),
        ],
        out_specs=[
            pl.BlockSpec((bm, nc), lambda g, i: (jnp.where(g == 0, i, last), 0)),
            pl.BlockSpec((bm, nc), lambda g, i: (jnp.where(g == 0, 0, i), 0)),
        ],
        out_shape=[
            jax.ShapeDtypeStruct((m, nc), _F32),
            jax.ShapeDtypeStruct((m, nc), _F32),
        ],
    )(a1, a2, z1, p2, o2t, b.reshape(1, nc))


def _att_kernel(p1_ref, p2_ref, z_ref, o2t_ref, *, scale, d, nc):
    i = pl.program_id(0)

    @pl.when(i == 0)
    def _():
        o2t_ref[...] = jnp.zeros_like(o2t_ref)

    q = p1_ref[:, :d]
    k = p2_ref[:, :d]
    u2 = p2_ref[:, d + nc:]
    v1 = p1_ref[:, d:d + nc]
    u1 = p1_ref[:, d + nc:]
    s = _dot(q, k, tb=True) * scale
    mx = jnp.max(s, axis=1, keepdims=True)
    e = jnp.exp(s - mx)
    ls = jnp.sum(e, axis=1, keepdims=True)
    p = (e / ls).astype(_BF16)
    z_ref[...] = (v1.astype(_F32) + _dot(p, u2)).astype(z_ref.dtype)
    dn = (((0,), (0,)), ((), ()))
    o2t_ref[...] += jax.lax.dot_general(u1, p, dn,
                                        preferred_element_type=_F32)


def _att(p1, p2, d, nc, bq=512):
    m, w = p1.shape
    bq = min(bq, m)
    kern = functools.partial(_att_kernel, scale=1.0 / (d ** 0.5), d=d, nc=nc)
    return pl.pallas_call(
        kern,
        grid=(m // bq,),
        in_specs=[
            pl.BlockSpec((bq, w), lambda i: (i, 0)),
            pl.BlockSpec((m, w), lambda i: (0, 0)),
        ],
        out_specs=[
            pl.BlockSpec((bq, nc), lambda i: (i, 0)),
            pl.BlockSpec((nc, m), lambda i: (0, 0)),
        ],
        out_shape=[
            jax.ShapeDtypeStruct((m, nc), _FP8),
            jax.ShapeDtypeStruct((nc, m), _F32),
        ],
    )(p1, p2)


def kernel(x1, adj1, x2, adj2, W1, b1, W2a, b2a, W2b, b2b, Wq, Wk, W3, b3):
    d = Wq.shape[1]
    nc = W3.shape[1]
    W3a, W3b = W3[:d], W3[d:]
    wcat1 = jnp.concatenate([Wq, W3a, W3b], axis=1).astype(_BF16)
    wcat2 = jnp.concatenate([Wk, W3a, W3b], axis=1).astype(_BF16)

    w1b = W1.astype(_BF16)
    s1, a1 = _gcn_cast(adj1, x1, w1b, b1, W2a.astype(_BF16))
    s2, a2 = _gcn_cast(adj2, x2, w1b, b1, W2a.astype(_BF16))
    s1, s2 = _gcn2(a1, a2, s1, s2, b2a, W2b.astype(_BF16))
    p1, p2 = _gcn_proj2(a1, jnp.concatenate([s1, s2], axis=1),
                        jnp.concatenate([b2b, b2b]), wcat1, wcat2)

    z1, o2t = _att(p1, p2, d, nc)

    rec1, rec2 = _final2(a1, a2, z1, p2, o2t, b3, d, nc)
    return (rec1, rec2)


# speedup vs baseline: 1.0247x; 1.0247x over previous
import functools

import jax
import jax.numpy as jnp
from jax.experimental import pallas as pl

_F32 = jnp.float32
_BF16 = jnp.bfloat16
_FP8 = jnp.float8_e4m3fn
_ADJ_SCALE = 4096.0
_INV_ADJ_SCALE = 1.0 / 4096.0


def _dot(a, b, tb=False):
    dn = (((1,), (1,) if tb else (0,)), ((), ()))
    return jax.lax.dot_general(a, b, dn, preferred_element_type=_F32)


def _mm_kernel(x_ref, w_ref, o_ref):
    o_ref[...] = _dot(x_ref[...], w_ref[...]).astype(o_ref.dtype)


def _mm(x, w, bm, out_dtype=_BF16):
    m, kdim = x.shape
    bm = min(bm, m)
    n = w.shape[1]
    return pl.pallas_call(
        _mm_kernel,
        grid=(m // bm,),
        in_specs=[
            pl.BlockSpec((bm, kdim), lambda i: (i, 0)),
            pl.BlockSpec((kdim, n), lambda i: (0, 0)),
        ],
        out_specs=pl.BlockSpec((bm, n), lambda i: (i, 0)),
        out_shape=jax.ShapeDtypeStruct((m, n), out_dtype),
    )(x, w)


def _gcn_cast_kernel(adj_ref, x_ref, w0_ref, b_ref, wn_ref, o_ref, a8_ref,
                     sup_ref):
    @pl.when(pl.program_id(0) == 0)
    def _():
        sup_ref[...] = _dot(x_ref[...].astype(_BF16),
                            w0_ref[...]).astype(_FP8)

    a = (adj_ref[...] * _ADJ_SCALE).astype(_FP8)
    a8_ref[...] = a
    acc = _dot(a, sup_ref[...]) * _INV_ADJ_SCALE
    z = jax.nn.sigmoid(acc + b_ref[...])
    o_ref[...] = _dot(z.astype(_BF16), wn_ref[...]).astype(o_ref.dtype)


def _gcn_cast(adj, x, w0, b, w_next, bm=1024, out_dtype=_FP8):
    from jax.experimental.pallas import tpu as pltpu
    m, kdim = adj.shape
    bm = min(bm, m)
    nfeat = x.shape[1]
    n = w0.shape[1]
    n_out = w_next.shape[1]
    return pl.pallas_call(
        _gcn_cast_kernel,
        grid=(m // bm,),
        in_specs=[
            pl.BlockSpec((bm, kdim), lambda i: (i, 0)),
            pl.BlockSpec((m, nfeat), lambda i: (0, 0)),
            pl.BlockSpec((nfeat, n), lambda i: (0, 0)),
            pl.BlockSpec((1, n), lambda i: (0, 0)),
            pl.BlockSpec((n, n_out), lambda i: (0, 0)),
        ],
        out_specs=[
            pl.BlockSpec((bm, n_out), lambda i: (i, 0)),
            pl.BlockSpec((bm, kdim), lambda i: (i, 0)),
        ],
        out_shape=[
            jax.ShapeDtypeStruct((m, n_out), out_dtype),
            jax.ShapeDtypeStruct((m, kdim), _FP8),
        ],
        scratch_shapes=[pltpu.VMEM((m, n), _FP8)],
    )(adj, x, w0, b.reshape(1, n), w_next)


def _gcn_kernel(adj_ref, sup_ref, b_ref, o_ref):
    z = _dot(adj_ref[...], sup_ref[...]) * _INV_ADJ_SCALE + b_ref[...]
    o_ref[...] = jax.nn.sigmoid(z).astype(o_ref.dtype)


def _gcn_proj_kernel(adj_ref, sup_ref, b_ref, wn_ref, o_ref):
    acc = _dot(adj_ref[...], sup_ref[...]) * _INV_ADJ_SCALE
    z = jax.nn.sigmoid(acc + b_ref[...])
    o_ref[...] = _dot(z.astype(_BF16), wn_ref[...]).astype(o_ref.dtype)


def _gcn(adj, sup, b, w_next=None, bm=1024, out_dtype=_BF16):
    m, kdim = adj.shape
    bm = min(bm, m)
    n = sup.shape[1]
    b2 = b.reshape(1, n)
    if w_next is None:
        kern, n_out, ops = _gcn_kernel, n, (adj, sup, b2)
        extra = []
    else:
        n_out = w_next.shape[1]
        kern, ops = _gcn_proj_kernel, (adj, sup, b2, w_next)
        extra = [pl.BlockSpec((n, n_out), lambda i: (0, 0))]
    return pl.pallas_call(
        kern,
        grid=(m // bm,),
        in_specs=[
            pl.BlockSpec((bm, kdim), lambda i: (i, 0)),
            pl.BlockSpec((kdim, n), lambda i: (0, 0)),
            pl.BlockSpec((1, n), lambda i: (0, 0)),
        ] + extra,
        out_specs=pl.BlockSpec((bm, n_out), lambda i: (i, 0)),
        out_shape=jax.ShapeDtypeStruct((m, n_out), out_dtype),
    )(*ops)


def _gcn_add_kernel(adj_ref, sa_ref, sb_ref, b_ref, o_ref):
    sup = (sa_ref[...].astype(_F32) + sb_ref[...]).astype(_FP8)
    z = _dot(adj_ref[...], sup) * _INV_ADJ_SCALE + b_ref[...]
    o_ref[...] = jax.nn.sigmoid(z).astype(o_ref.dtype)


def _gcn_add(adj, sup_a, sup_b, b, bm=1024):
    m, kdim = adj.shape
    bm = min(bm, m)
    n = sup_a.shape[1]
    return pl.pallas_call(
        _gcn_add_kernel,
        grid=(m // bm,),
        in_specs=[
            pl.BlockSpec((bm, kdim), lambda i: (i, 0)),
            pl.BlockSpec((kdim, n), lambda i: (0, 0)),
            pl.BlockSpec((kdim, n), lambda i: (0, 0)),
            pl.BlockSpec((1, n), lambda i: (0, 0)),
        ],
        out_specs=pl.BlockSpec((bm, n), lambda i: (i, 0)),
        out_shape=jax.ShapeDtypeStruct((m, n), _F32),
    )(adj, sup_a, sup_b, b.reshape(1, n))


def _gcn_proj2_kernel(adj_ref, sup_ref, b_ref, w1_ref, w2_ref, o1_ref,
                      o2_ref, *, n):
    acc = _dot(adj_ref[...], sup_ref[...]) * _INV_ADJ_SCALE
    z = jax.nn.sigmoid(acc + b_ref[...])
    zb = z.astype(_BF16)
    o1_ref[...] = _dot(zb[:, :n], w1_ref[...]).astype(o1_ref.dtype)
    o2_ref[...] = _dot(zb[:, n:], w2_ref[...]).astype(o2_ref.dtype)


def _gcn_proj2(adj, sup, b, w1, w2, bm=1024):
    m, kdim = adj.shape
    bm = min(bm, m)
    n2 = sup.shape[1]
    n = n2 // 2
    n_out = w1.shape[1]
    kern = functools.partial(_gcn_proj2_kernel, n=n)
    return pl.pallas_call(
        kern,
        grid=(m // bm,),
        in_specs=[
            pl.BlockSpec((bm, kdim), lambda i: (i, 0)),
            pl.BlockSpec((kdim, n2), lambda i: (0, 0)),
            pl.BlockSpec((1, n2), lambda i: (0, 0)),
            pl.BlockSpec((n, n_out), lambda i: (0, 0)),
            pl.BlockSpec((n, n_out), lambda i: (0, 0)),
        ],
        out_specs=[
            pl.BlockSpec((bm, n_out), lambda i: (i, 0)),
            pl.BlockSpec((bm, n_out), lambda i: (i, 0)),
        ],
        out_shape=[
            jax.ShapeDtypeStruct((m, n_out), _BF16),
            jax.ShapeDtypeStruct((m, n_out), _BF16),
        ],
    )(adj, sup, b.reshape(1, n2), w1, w2)


def _att_kernel(p1_ref, p2_ref, z_ref, o2t_ref, *, scale, d, nc):
    i = pl.program_id(0)

    @pl.when(i == 0)
    def _():
        o2t_ref[...] = jnp.zeros_like(o2t_ref)

    q = p1_ref[:, :d]
    k = p2_ref[:, :d]
    u2 = p2_ref[:, d + nc:]
    v1 = p1_ref[:, d:d + nc]
    u1 = p1_ref[:, d + nc:]
    s = _dot(q, k, tb=True) * scale
    mx = jnp.max(s, axis=1, keepdims=True)
    e = jnp.exp(s - mx)
    ls = jnp.sum(e, axis=1, keepdims=True)
    p = (e / ls).astype(_BF16)
    z_ref[...] = (v1.astype(_F32) + _dot(p, u2)).astype(z_ref.dtype)
    dn = (((0,), (0,)), ((), ()))
    o2t_ref[...] += jax.lax.dot_general(u1, p, dn,
                                        preferred_element_type=_F32)


def _att(p1, p2, d, nc, bq=512):
    m, w = p1.shape
    bq = min(bq, m)
    kern = functools.partial(_att_kernel, scale=1.0 / (d ** 0.5), d=d, nc=nc)
    return pl.pallas_call(
        kern,
        grid=(m // bq,),
        in_specs=[
            pl.BlockSpec((bq, w), lambda i: (i, 0)),
            pl.BlockSpec((m, w), lambda i: (0, 0)),
        ],
        out_specs=[
            pl.BlockSpec((bq, nc), lambda i: (i, 0)),
            pl.BlockSpec((nc, m), lambda i: (0, 0)),
        ],
        out_shape=[
            jax.ShapeDtypeStruct((m, nc), _FP8),
            jax.ShapeDtypeStruct((nc, m), _F32),
        ],
    )(p1, p2)


def kernel(x1, adj1, x2, adj2, W1, b1, W2a, b2a, W2b, b2b, Wq, Wk, W3, b3):
    d = Wq.shape[1]
    nc = W3.shape[1]
    W3a, W3b = W3[:d], W3[d:]
    wcat1 = jnp.concatenate([Wq, W3a, W3b], axis=1).astype(_BF16)
    wcat2 = jnp.concatenate([Wk, W3a, W3b], axis=1).astype(_BF16)

    w1b = W1.astype(_BF16)
    s1, a1 = _gcn_cast(adj1, x1, w1b, b1, W2a.astype(_BF16))
    s2, a2 = _gcn_cast(adj2, x2, w1b, b1, W2a.astype(_BF16))
    s1 = _gcn(a1, s1, b2a, W2b.astype(_BF16), out_dtype=_FP8)
    s2 = _gcn(a2, s2, b2a, W2b.astype(_BF16), out_dtype=_FP8)
    p1, p2 = _gcn_proj2(a1, jnp.concatenate([s1, s2], axis=1),
                        jnp.concatenate([b2b, b2b]), wcat1, wcat2)

    z1, o2t = _att(p1, p2, d, nc)

    rec1 = _gcn(a1, z1, b3, out_dtype=_F32)
    rec2 = _gcn_add(a2, p2[:, d:d + nc], o2t.T, b3)
    return (rec1, rec2)
